# baseline (speedup 1.0000x reference)
import functools

import jax
import jax.numpy as jnp
from jax import lax
from jax.experimental import pallas as pl
from jax.experimental.pallas import tpu as pltpu
from jax.experimental.pallas import tpu_sc as plsc

_N = 10000
_E = 320000
_D = 128
_G = 64

_NC = 2
_NS = 16
_L = 16
_NW = _NC * _NS

_EW = _E // _NW
_C = 128
_NFULL = _EW // _C
_CT = _EW - _NFULL * _C
_NP = 10240
_ROWS_PER_SUB = _NP // _NS
_N_STAGE = _ROWS_PER_SUB // _C


def _make_sc_agg(d):
    mesh = plsc.VectorSubcoreMesh(core_axis_name="c", subcore_axis_name="s")

    @functools.partial(
        pl.kernel,
        mesh=mesh,
        out_type=jax.ShapeDtypeStruct((_NC, _NP, d), jnp.float32),
        scratch_types=[
            pltpu.VMEM((_EW,), jnp.int32),
            pltpu.VMEM((_C,), jnp.int32),
            pltpu.VMEM((_C,), jnp.int32),
            pltpu.VMEM((_CT,), jnp.int32),
            pltpu.VMEM((_C, d), jnp.float32),
            pltpu.VMEM((_C, d), jnp.float32),
            pltpu.VMEM_SHARED((_NP, d), jnp.float32),
            pltpu.SemaphoreType.DMA,
            pltpu.SemaphoreType.DMA,
            pltpu.SemaphoreType.DMA,
            pltpu.SemaphoreType.DMA,
            pltpu.SemaphoreType.DMA,
            pltpu.SemaphoreType.DMA,
        ],
    )
    def agg(x_hbm, src_hbm, dst_hbm, out_hbm,
            src_v, didx0, didx1, didxt, buf0, buf1, acc_sh,
            semg0, semg1, sems0, sems1, semi0, semi1):
        cid = lax.axis_index("c")
        sid = lax.axis_index("s")
        wid = cid * _NS + sid
        row0 = sid * _ROWS_PER_SUB

        pltpu.sync_copy(src_hbm.at[pl.ds(wid * _EW, _EW)], src_v)

        def zrow(r, carry):
            for c8 in range(d // _L):
                buf0[r, pl.ds(c8 * _L, _L)] = jnp.zeros((_L,), jnp.float32)
            return carry

        lax.fori_loop(0, _C, zrow, 0)

        def zcopy(j, carry):
            pltpu.sync_copy(buf0, acc_sh.at[pl.ds(row0 + j * _C, _C)])
            return carry

        lax.fori_loop(0, _N_STAGE, zcopy, 0)
        plsc.subcore_barrier()

        bufs = (buf0, buf1)
        didxs = (didx0, didx1)
        semgs = (semg0, semg1)
        semss = (sems0, sems1)
        semis = (semi0, semi1)
        n_pairs = _NFULL // 2
        ebase = wid * _EW

        def gather(p, b):
            pltpu.async_copy(
                x_hbm.at[src_v.at[pl.ds(p * _C, _C)]], bufs[b], semgs[b])

        def wait_gather(b):
            pltpu.make_async_copy(
                x_hbm.at[pl.ds(0, _C)], bufs[b], semgs[b]).wait()

        def fetch_didx(p, b):
            pltpu.async_copy(
                dst_hbm.at[pl.ds(ebase + p * _C, _C)], didxs[b], semis[b])

        def wait_didx(b):
            pltpu.make_async_copy(
                dst_hbm.at[pl.ds(0, _C)], didxs[b], semis[b]).wait()

        def scatter(b):
            pltpu.async_copy(bufs[b], acc_sh.at[didxs[b]], semss[b], add=True)

        def wait_scatter(b):
            pltpu.make_async_copy(
                x_hbm.at[pl.ds(0, _C)], bufs[b], semss[b]).wait()

        gather(0, 0)
        fetch_didx(0, 0)
        gather(1, 1)
        fetch_didx(1, 1)

        def body(q, carry):
            p0 = 2 * q
            wait_gather(0)
            wait_didx(0)
            scatter(0)
            wait_gather(1)
            wait_didx(1)
            scatter(1)

            @pl.when(q < n_pairs - 1)
            def _():
                wait_scatter(0)
                gather(p0 + 2, 0)
                fetch_didx(p0 + 2, 0)
                wait_scatter(1)
                gather(p0 + 3, 1)
                fetch_didx(p0 + 3, 1)

            return carry

        lax.fori_loop(0, n_pairs, body, 0)

        wait_scatter(0)
        pltpu.async_copy(
            x_hbm.at[src_v.at[pl.ds(_NFULL * _C, _CT)]],
            buf0.at[pl.ds(0, _CT)], semg0)
        pltpu.async_copy(
            dst_hbm.at[pl.ds(ebase + _NFULL * _C, _CT)], didxt, semi0)
        wait_scatter(1)
        pltpu.make_async_copy(
            x_hbm.at[pl.ds(0, _CT)], buf0.at[pl.ds(0, _CT)], semg0).wait()
        pltpu.make_async_copy(
            dst_hbm.at[pl.ds(0, _CT)], didxt, semi0).wait()
        pltpu.async_copy(
            buf0.at[pl.ds(0, _CT)], acc_sh.at[didxt], sems0, add=True)
        pltpu.make_async_copy(
            x_hbm.at[pl.ds(0, _CT)], buf0.at[pl.ds(0, _CT)], sems0).wait()
        plsc.subcore_barrier()

        def out_body(j, carry):
            r = row0 + j * _C
            pltpu.sync_copy(acc_sh.at[pl.ds(r, _C)], buf0)
            pltpu.sync_copy(buf0, out_hbm.at[cid, pl.ds(r, _C)])
            return carry

        lax.fori_loop(0, _N_STAGE, out_body, 0)

    return agg


_sc_agg = _make_sc_agg(_D)


def _mlp(x, Wa, ba, g, be, Wb, bb):
    h = jnp.dot(x, Wa, preferred_element_type=jnp.float32) + ba
    m = jnp.mean(h, axis=0, keepdims=True)
    v = jnp.mean((h - m) ** 2, axis=0, keepdims=True)
    h = (h - m) * lax.rsqrt(v + 1e-5) * g + be
    h = jnp.maximum(h, 0.0)
    h = jnp.dot(h, Wb, preferred_element_type=jnp.float32) + bb
    return jnp.maximum(h, 0.0)


def _mlp_kernel(x_ref, agg_ref, Wa_ref, ba_ref, g_ref, be_ref, Wb_ref,
                bb_ref, out_ref):
    x = x_ref[...] + agg_ref[0, :_N, :] + agg_ref[1, :_N, :]
    out_ref[...] = _mlp(x, Wa_ref[...], ba_ref[...], g_ref[...],
                        be_ref[...], Wb_ref[...], bb_ref[...])


def _mlp_pool_kernel(x_ref, agg_ref, batch_ref, Wa_ref, ba_ref, g_ref,
                     be_ref, Wb_ref, bb_ref, Wl_ref, bl_ref, out_ref):
    x = x_ref[...] + agg_ref[0, :_N, :] + agg_ref[1, :_N, :]
    h = _mlp(x, Wa_ref[...], ba_ref[...], g_ref[...], be_ref[...],
             Wb_ref[...], bb_ref[...])
    onehot = (batch_ref[...] ==
              lax.broadcasted_iota(jnp.int32, (_G, _N), 0)).astype(jnp.float32)
    pooled = jnp.dot(onehot, h, preferred_element_type=jnp.float32)
    out_ref[...] = (jnp.dot(pooled, Wl_ref[...],
                            preferred_element_type=jnp.float32) + bl_ref[...])


def kernel(x, edge_index, batch, W1a, b1a, g1, be1, W1b, b1b,
           W2a, b2a, g2, be2, W2b, b2b, Wl, bl):
    src = edge_index[0]
    dst = edge_index[1]

    agg1 = _sc_agg(x, src, dst)
    h1 = pl.pallas_call(
        _mlp_kernel,
        out_shape=jax.ShapeDtypeStruct((_N, _D), jnp.float32),
    )(x, agg1, W1a, b1a.reshape(1, -1), g1.reshape(1, -1),
      be1.reshape(1, -1), W1b, b1b.reshape(1, -1))

    agg2 = _sc_agg(h1, src, dst)
    out = pl.pallas_call(
        _mlp_pool_kernel,
        out_shape=jax.ShapeDtypeStruct((_G, 1), jnp.float32),
    )(h1, agg2, batch.reshape(1, -1), W2a, b2a.reshape(1, -1),
      g2.reshape(1, -1), be2.reshape(1, -1), W2b, b2b.reshape(1, -1),
      Wl, bl.reshape(1, -1))
    return out


# speedup vs baseline: 1.1935x; 1.1935x over previous
import functools

import jax
import jax.numpy as jnp
from jax import lax
from jax.experimental import pallas as pl
from jax.experimental.pallas import tpu as pltpu
from jax.experimental.pallas import tpu_sc as plsc

_N = 10000
_E = 320000
_D = 128
_G = 64

_NC = 2
_NS = 16
_L = 16
_NW = _NC * _NS

_EW = _E // _NW
_C = 80
_NCHUNK = _EW // _C
_NP = 10240
_ROWS_PER_SUB = _NP // _NS
_N_STAGE = _ROWS_PER_SUB // _C


def _make_sc_agg(d):
    mesh = plsc.VectorSubcoreMesh(core_axis_name="c", subcore_axis_name="s")

    @functools.partial(
        pl.kernel,
        mesh=mesh,
        out_type=jax.ShapeDtypeStruct((_NC, _NP, d), jnp.float32),
        scratch_types=[
            pltpu.VMEM((_EW,), jnp.int32),
            pltpu.VMEM((_EW,), jnp.int32),
            pltpu.VMEM((_C,), jnp.int32),
            pltpu.VMEM((_C, d), jnp.float32),
            pltpu.VMEM((_C, d), jnp.float32),
            pltpu.VMEM_SHARED((_NP, d), jnp.float32),
            pltpu.SemaphoreType.DMA,
            pltpu.SemaphoreType.DMA,
            pltpu.SemaphoreType.DMA,
            pltpu.SemaphoreType.DMA,
        ],
    )
    def agg(x_hbm, src_hbm, dst_hbm, out_hbm,
            src_v, dst_v, didx, buf0, buf1, acc_sh, sem0, sem1, semi, semz):
        cid = lax.axis_index("c")
        sid = lax.axis_index("s")
        wid = cid * _NS + sid
        row0 = sid * _ROWS_PER_SUB

        pltpu.async_copy(src_hbm.at[pl.ds(wid * _EW, _EW)], src_v, semi)
        pltpu.async_copy(dst_hbm.at[pl.ds(wid * _EW, _EW)], dst_v, semi)

        def zrow(r, carry):
            for c8 in range(d // _L):
                buf0[r, pl.ds(c8 * _L, _L)] = jnp.zeros((_L,), jnp.float32)
            return carry

        lax.fori_loop(0, _C, zrow, 0)

        def zcopy(j, carry):
            pltpu.async_copy(buf0, acc_sh.at[pl.ds(row0 + j * _C, _C)], semz)
            return carry

        lax.fori_loop(0, _N_STAGE, zcopy, 0)

        pltpu.make_async_copy(
            src_hbm.at[pl.ds(0, _EW)], src_v, semi).wait()
        pltpu.make_async_copy(
            dst_hbm.at[pl.ds(0, _EW)], dst_v, semi).wait()

        def zdrain(j, carry):
            pltpu.make_async_copy(
                buf0, acc_sh.at[pl.ds(row0, _C)], semz).wait()
            return carry

        lax.fori_loop(0, _N_STAGE, zdrain, 0)
        plsc.subcore_barrier()

        bufs = (buf0, buf1)
        sems = (sem0, sem1)
        n_pairs = _NCHUNK // 2

        def gather(p, b):
            pltpu.async_copy(
                x_hbm.at[src_v.at[pl.ds(p * _C, _C)]], bufs[b], sems[b])

        def wait_gather(b):
            pltpu.make_async_copy(
                x_hbm.at[pl.ds(0, _C)], bufs[b], sems[b]).wait()

        def scatter(p, b):
            for k in range(_C // _L):
                didx[pl.ds(k * _L, _L)] = dst_v[pl.ds(p * _C + k * _L, _L)]
            pltpu.sync_copy(bufs[b], acc_sh.at[didx], add=True)

        gather(0, 0)

        def body(q, carry):
            p0 = 2 * q
            gather(p0 + 1, 1)
            wait_gather(0)
            scatter(p0, 0)
            gather(p0 + 2, 0)
            wait_gather(1)
            scatter(p0 + 1, 1)
            return carry

        lax.fori_loop(0, n_pairs, body, 0)
        wait_gather(0)
        scatter(_NCHUNK - 1, 0)
        plsc.subcore_barrier()

        def out_body(q, carry):
            for b in range(2):
                r = row0 + (2 * q + b) * _C
                pltpu.sync_copy(acc_sh.at[pl.ds(r, _C)], bufs[b])
                pltpu.async_copy(bufs[b], out_hbm.at[cid, pl.ds(r, _C)], sems[b])
            for b in range(2):
                r = row0 + (2 * q + b) * _C
                pltpu.make_async_copy(
                    bufs[b], out_hbm.at[cid, pl.ds(r, _C)], sems[b]).wait()
            return carry

        lax.fori_loop(0, _N_STAGE // 2, out_body, 0)

    return agg


_sc_agg = _make_sc_agg(_D)


def _mlp(x, Wa, ba, g, be, Wb, bb):
    h = jnp.dot(x, Wa, preferred_element_type=jnp.float32) + ba
    m = jnp.mean(h, axis=0, keepdims=True)
    v = jnp.mean((h - m) ** 2, axis=0, keepdims=True)
    h = (h - m) * lax.rsqrt(v + 1e-5) * g + be
    h = jnp.maximum(h, 0.0)
    h = jnp.dot(h, Wb, preferred_element_type=jnp.float32) + bb
    return jnp.maximum(h, 0.0)


def _mlp_kernel(x_ref, agg_ref, Wa_ref, ba_ref, g_ref, be_ref, Wb_ref,
                bb_ref, out_ref):
    x = x_ref[...] + agg_ref[0, :_N, :] + agg_ref[1, :_N, :]
    out_ref[...] = _mlp(x, Wa_ref[...], ba_ref[...], g_ref[...],
                        be_ref[...], Wb_ref[...], bb_ref[...])


def _mlp_pool_kernel(x_ref, agg_ref, batch_ref, Wa_ref, ba_ref, g_ref,
                     be_ref, Wb_ref, bb_ref, Wl_ref, bl_ref, out_ref):
    x = x_ref[...] + agg_ref[0, :_N, :] + agg_ref[1, :_N, :]
    h = _mlp(x, Wa_ref[...], ba_ref[...], g_ref[...], be_ref[...],
             Wb_ref[...], bb_ref[...])
    onehot = (batch_ref[...] ==
              lax.broadcasted_iota(jnp.int32, (_G, _N), 0)).astype(jnp.float32)
    pooled = jnp.dot(onehot, h, preferred_element_type=jnp.float32)
    out_ref[...] = (jnp.dot(pooled, Wl_ref[...],
                            preferred_element_type=jnp.float32) + bl_ref[...])


def kernel(x, edge_index, batch, W1a, b1a, g1, be1, W1b, b1b,
           W2a, b2a, g2, be2, W2b, b2b, Wl, bl):
    src = edge_index[0]
    dst = edge_index[1]

    agg1 = _sc_agg(x, src, dst)
    h1 = pl.pallas_call(
        _mlp_kernel,
        out_shape=jax.ShapeDtypeStruct((_N, _D), jnp.float32),
    )(x, agg1, W1a, b1a.reshape(1, -1), g1.reshape(1, -1),
      be1.reshape(1, -1), W1b, b1b.reshape(1, -1))

    agg2 = _sc_agg(h1, src, dst)
    out = pl.pallas_call(
        _mlp_pool_kernel,
        out_shape=jax.ShapeDtypeStruct((_G, 1), jnp.float32),
    )(h1, agg2, batch.reshape(1, -1), W2a, b2a.reshape(1, -1),
      g2.reshape(1, -1), be2.reshape(1, -1), W2b, b2b.reshape(1, -1),
      Wl, bl.reshape(1, -1))
    return out


# speedup vs baseline: 1.4194x; 1.1892x over previous
import functools

import jax
import jax.numpy as jnp
from jax import lax
from jax.experimental import pallas as pl
from jax.experimental.pallas import tpu as pltpu
from jax.experimental.pallas import tpu_sc as plsc

_N = 10000
_E = 320000
_D = 128
_G = 64

_NC = 2
_NS = 16
_L = 16
_NW = _NC * _NS

_EW = _E // _NW
_C = 80
_NCHUNK = _EW // _C
_NP = 10240
_ROWS_PER_SUB = _NP // _NS
_N_STAGE = _ROWS_PER_SUB // _C


def _make_sc_agg(d):
    mesh = plsc.VectorSubcoreMesh(core_axis_name="c", subcore_axis_name="s")

    @functools.partial(
        pl.kernel,
        mesh=mesh,
        out_type=jax.ShapeDtypeStruct((_NC, _NP, d), jnp.float32),
        scratch_types=[
            pltpu.VMEM((_EW,), jnp.int32),
            pltpu.VMEM((_C,), jnp.int32),
            pltpu.VMEM((_C,), jnp.int32),
            pltpu.VMEM((_C,), jnp.int32),
            pltpu.VMEM((_C, d), jnp.float32),
            pltpu.VMEM((_C, d), jnp.float32),
            pltpu.VMEM((_C, d), jnp.float32),
            pltpu.VMEM_SHARED((_NP, d), jnp.float32),
            pltpu.SemaphoreType.DMA,
            pltpu.SemaphoreType.DMA,
            pltpu.SemaphoreType.DMA,
            pltpu.SemaphoreType.DMA,
            pltpu.SemaphoreType.DMA,
            pltpu.SemaphoreType.DMA,
            pltpu.SemaphoreType.DMA,
            pltpu.SemaphoreType.DMA,
        ],
    )
    def agg(x_hbm, src_hbm, dst_hbm, out_hbm,
            src_v, didx0, didx1, didx2, buf0, buf1, buf2, acc_sh,
            sem0, sem1, sem2, semd0, semd1, semd2, semi, semz):
        cid = lax.axis_index("c")
        sid = lax.axis_index("s")
        wid = cid * _NS + sid
        row0 = sid * _ROWS_PER_SUB

        pltpu.async_copy(src_hbm.at[pl.ds(wid * _EW, _EW)], src_v, semi)

        def zrow(r, carry):
            for c8 in range(d // _L):
                buf0[r, pl.ds(c8 * _L, _L)] = jnp.zeros((_L,), jnp.float32)
            return carry

        lax.fori_loop(0, _C, zrow, 0)

        def zcopy(j, carry):
            pltpu.async_copy(buf0, acc_sh.at[pl.ds(row0 + j * _C, _C)], semz)
            return carry

        lax.fori_loop(0, _N_STAGE, zcopy, 0)

        def zdrain(j, carry):
            pltpu.make_async_copy(
                buf0, acc_sh.at[pl.ds(row0, _C)], semz).wait()
            return carry

        lax.fori_loop(0, _N_STAGE, zdrain, 0)

        bufs = (buf0, buf1, buf2)
        sems = (sem0, sem1, sem2)
        didxs = (didx0, didx1, didx2)
        semds = (semd0, semd1, semd2)
        n_triples = _NCHUNK // 3
        ebase = wid * _EW

        def gather(p, b):
            pltpu.async_copy(
                x_hbm.at[src_v.at[pl.ds(p * _C, _C)]], bufs[b], sems[b])

        def wait_gather(b):
            pltpu.make_async_copy(
                x_hbm.at[pl.ds(0, _C)], bufs[b], sems[b]).wait()

        def fetch_didx(p, b):
            pltpu.async_copy(
                dst_hbm.at[pl.ds(ebase + p * _C, _C)], didxs[b], semds[b])

        def wait_didx(b):
            pltpu.make_async_copy(
                dst_hbm.at[pl.ds(0, _C)], didxs[b], semds[b]).wait()

        for j in range(3):
            fetch_didx(j, j)

        pltpu.make_async_copy(
            src_hbm.at[pl.ds(0, _EW)], src_v, semi).wait()
        plsc.subcore_barrier()

        for j in range(3):
            gather(j, j)

        def body(k, carry):
            p0 = 3 * k
            for j in range(3):
                wait_gather(j)
                wait_didx(j)
                pltpu.sync_copy(bufs[j], acc_sh.at[didxs[j]], add=True)
                if j == 2:
                    @pl.when(k < n_triples - 1)
                    def _():
                        fetch_didx(p0 + j + 3, j)
                        gather(p0 + j + 3, j)
                else:
                    fetch_didx(p0 + j + 3, j)
                    gather(p0 + j + 3, j)
            return carry

        lax.fori_loop(0, n_triples, body, 0)
        for j in range(2):
            wait_gather(j)
            wait_didx(j)
            pltpu.sync_copy(bufs[j], acc_sh.at[didxs[j]], add=True)
        plsc.subcore_barrier()

        def out_body(q, carry):
            for b in range(2):
                r = row0 + (2 * q + b) * _C
                pltpu.sync_copy(acc_sh.at[pl.ds(r, _C)], bufs[b])
                pltpu.async_copy(bufs[b], out_hbm.at[cid, pl.ds(r, _C)], sems[b])
            for b in range(2):
                r = row0 + (2 * q + b) * _C
                pltpu.make_async_copy(
                    bufs[b], out_hbm.at[cid, pl.ds(r, _C)], sems[b]).wait()
            return carry

        lax.fori_loop(0, _N_STAGE // 2, out_body, 0)

    return agg


_sc_agg = _make_sc_agg(_D)


def _mlp(x, Wa, ba, g, be, Wb, bb):
    h = jnp.dot(x, Wa, preferred_element_type=jnp.float32) + ba
    m = jnp.mean(h, axis=0, keepdims=True)
    v = jnp.mean((h - m) ** 2, axis=0, keepdims=True)
    h = (h - m) * lax.rsqrt(v + 1e-5) * g + be
    h = jnp.maximum(h, 0.0)
    h = jnp.dot(h, Wb, preferred_element_type=jnp.float32) + bb
    return jnp.maximum(h, 0.0)


def _mlp_kernel(x_ref, agg_ref, Wa_ref, ba_ref, g_ref, be_ref, Wb_ref,
                bb_ref, out_ref):
    x = x_ref[...] + agg_ref[0, :_N, :] + agg_ref[1, :_N, :]
    out_ref[...] = _mlp(x, Wa_ref[...], ba_ref[...], g_ref[...],
                        be_ref[...], Wb_ref[...], bb_ref[...])


def _mlp_pool_kernel(x_ref, agg_ref, batch_ref, Wa_ref, ba_ref, g_ref,
                     be_ref, Wb_ref, bb_ref, Wl_ref, bl_ref, out_ref):
    x = x_ref[...] + agg_ref[0, :_N, :] + agg_ref[1, :_N, :]
    h = _mlp(x, Wa_ref[...], ba_ref[...], g_ref[...], be_ref[...],
             Wb_ref[...], bb_ref[...])
    onehot = (batch_ref[...] ==
              lax.broadcasted_iota(jnp.int32, (_G, _N), 0)).astype(jnp.float32)
    pooled = jnp.dot(onehot, h, preferred_element_type=jnp.float32)
    out_ref[...] = (jnp.dot(pooled, Wl_ref[...],
                            preferred_element_type=jnp.float32) + bl_ref[...])


def kernel(x, edge_index, batch, W1a, b1a, g1, be1, W1b, b1b,
           W2a, b2a, g2, be2, W2b, b2b, Wl, bl):
    src = edge_index[0]
    dst = edge_index[1]

    agg1 = _sc_agg(x, src, dst)
    h1 = pl.pallas_call(
        _mlp_kernel,
        out_shape=jax.ShapeDtypeStruct((_N, _D), jnp.float32),
    )(x, agg1, W1a, b1a.reshape(1, -1), g1.reshape(1, -1),
      be1.reshape(1, -1), W1b, b1b.reshape(1, -1))

    agg2 = _sc_agg(h1, src, dst)
    out = pl.pallas_call(
        _mlp_pool_kernel,
        out_shape=jax.ShapeDtypeStruct((_G, 1), jnp.float32),
    )(h1, agg2, batch.reshape(1, -1), W2a, b2a.reshape(1, -1),
      g2.reshape(1, -1), be2.reshape(1, -1), W2b, b2b.reshape(1, -1),
      Wl, bl.reshape(1, -1))
    return out


# speedup vs baseline: 1.4675x; 1.0339x over previous
import functools

import jax
import jax.numpy as jnp
from jax import lax
from jax.experimental import pallas as pl
from jax.experimental.pallas import tpu as pltpu
from jax.experimental.pallas import tpu_sc as plsc

_N = 10000
_E = 320000
_D = 128
_G = 64

_NC = 2
_NS = 16
_L = 16
_NW = _NC * _NS

_EW = _E // _NW
_C = 80
_NCHUNK = _EW // _C
_NP = 10240
_ROWS_PER_SUB = _NP // _NS
_N_STAGE = _ROWS_PER_SUB // _C


def _make_sc_agg(d):
    mesh = plsc.VectorSubcoreMesh(core_axis_name="c", subcore_axis_name="s")

    @functools.partial(
        pl.kernel,
        mesh=mesh,
        out_type=jax.ShapeDtypeStruct((_NC, _NP, d), jnp.float32),
        scratch_types=(
            [pltpu.VMEM((_C,), jnp.int32)] * 8
            + [pltpu.VMEM((_C, d), jnp.float32)] * 4
            + [pltpu.VMEM_SHARED((_NP, d), jnp.float32)]
            + [pltpu.SemaphoreType.DMA] * 13
        ),
    )
    def agg(x_hbm, src_hbm, dst_hbm, out_hbm,
            si0, si1, si2, si3, di0, di1, di2, di3,
            b0, b1, b2, b3, acc_sh,
            sg0, sg1, sg2, sg3, ss0, ss1, ss2, ss3,
            sd0, sd1, sd2, sd3, semz):
        sidxs = (si0, si1, si2, si3)
        didxs = (di0, di1, di2, di3)
        bufs = (b0, b1, b2, b3)
        semgs = (sg0, sg1, sg2, sg3)
        semss = (ss0, ss1, ss2, ss3)
        semds = (sd0, sd1, sd2, sd3)
        cid = lax.axis_index("c")
        sid = lax.axis_index("s")
        wid = cid * _NS + sid
        row0 = sid * _ROWS_PER_SUB
        ebase = wid * _EW
        buf0 = bufs[0]

        def zrow(r, carry):
            for c8 in range(d // _L):
                buf0[r, pl.ds(c8 * _L, _L)] = jnp.zeros((_L,), jnp.float32)
            return carry

        lax.fori_loop(0, _C, zrow, 0)

        def zcopy(j, carry):
            pltpu.async_copy(buf0, acc_sh.at[pl.ds(row0 + j * _C, _C)], semz)
            return carry

        lax.fori_loop(0, _N_STAGE, zcopy, 0)

        def zdrain(j, carry):
            pltpu.make_async_copy(
                buf0, acc_sh.at[pl.ds(row0, _C)], semz).wait()
            return carry

        n_quads = _NCHUNK // 4

        def gather(b):
            pltpu.async_copy(x_hbm.at[sidxs[b]], bufs[b], semgs[b])

        def wait_gather(b):
            pltpu.make_async_copy(
                x_hbm.at[pl.ds(0, _C)], bufs[b], semgs[b]).wait()

        def fetch_sidx(p, b):
            pltpu.async_copy(
                src_hbm.at[pl.ds(ebase + p * _C, _C)], sidxs[b], semss[b])

        def wait_sidx(b):
            pltpu.make_async_copy(
                src_hbm.at[pl.ds(0, _C)], sidxs[b], semss[b]).wait()

        def fetch_didx(p, b):
            pltpu.async_copy(
                dst_hbm.at[pl.ds(ebase + p * _C, _C)], didxs[b], semds[b])

        def wait_didx(b):
            pltpu.make_async_copy(
                dst_hbm.at[pl.ds(0, _C)], didxs[b], semds[b]).wait()

        for j in range(4):
            fetch_sidx(j, j)
            fetch_didx(j, j)

        lax.fori_loop(0, _N_STAGE, zdrain, 0)
        plsc.subcore_barrier()

        for j in range(4):
            wait_sidx(j)
            gather(j)
            fetch_sidx(j + 4, j)

        def body(k, carry):
            p0 = 4 * k
            for j in range(4):
                wait_gather(j)
                wait_didx(j)
                pltpu.sync_copy(bufs[j], acc_sh.at[didxs[j]], add=True)

                def refill(jj=j, pp=p0):
                    fetch_didx(pp + jj + 4, jj)
                    wait_sidx(jj)
                    gather(jj)
                    return None

                def prefetch(jj=j, pp=p0):
                    fetch_sidx(pp + jj + 8, jj)
                    return None

                if j == 0:
                    refill()
                    pl.when(k < n_quads - 1)(prefetch)
                else:
                    pl.when(k < n_quads - 1)(refill)
                    pl.when(k < n_quads - 2)(prefetch)
            return carry

        lax.fori_loop(0, n_quads, body, 0)
        wait_gather(0)
        wait_didx(0)
        pltpu.sync_copy(bufs[0], acc_sh.at[didxs[0]], add=True)
        plsc.subcore_barrier()

        def out_body(q, carry):
            for b in range(2):
                r = row0 + (2 * q + b) * _C
                pltpu.sync_copy(acc_sh.at[pl.ds(r, _C)], bufs[b])
                pltpu.async_copy(bufs[b], out_hbm.at[cid, pl.ds(r, _C)], semgs[b])
            for b in range(2):
                r = row0 + (2 * q + b) * _C
                pltpu.make_async_copy(
                    bufs[b], out_hbm.at[cid, pl.ds(r, _C)], semgs[b]).wait()
            return carry

        lax.fori_loop(0, _N_STAGE // 2, out_body, 0)

    return agg


_sc_agg = _make_sc_agg(_D)


def _mlp(x, Wa, ba, g, be, Wb, bb):
    h = jnp.dot(x, Wa, preferred_element_type=jnp.float32) + ba
    m = jnp.mean(h, axis=0, keepdims=True)
    v = jnp.mean((h - m) ** 2, axis=0, keepdims=True)
    h = (h - m) * lax.rsqrt(v + 1e-5) * g + be
    h = jnp.maximum(h, 0.0)
    h = jnp.dot(h, Wb, preferred_element_type=jnp.float32) + bb
    return jnp.maximum(h, 0.0)


def _mlp_kernel(x_ref, agg_ref, Wa_ref, ba_ref, g_ref, be_ref, Wb_ref,
                bb_ref, out_ref):
    x = x_ref[...] + agg_ref[0, :_N, :] + agg_ref[1, :_N, :]
    out_ref[...] = _mlp(x, Wa_ref[...], ba_ref[...], g_ref[...],
                        be_ref[...], Wb_ref[...], bb_ref[...])


def _mlp_pool_kernel(x_ref, agg_ref, batch_ref, Wa_ref, ba_ref, g_ref,
                     be_ref, Wb_ref, bb_ref, Wl_ref, bl_ref, out_ref):
    x = x_ref[...] + agg_ref[0, :_N, :] + agg_ref[1, :_N, :]
    h = _mlp(x, Wa_ref[...], ba_ref[...], g_ref[...], be_ref[...],
             Wb_ref[...], bb_ref[...])
    onehot = (batch_ref[...] ==
              lax.broadcasted_iota(jnp.int32, (_G, _N), 0)).astype(jnp.float32)
    pooled = jnp.dot(onehot, h, preferred_element_type=jnp.float32)
    out_ref[...] = (jnp.dot(pooled, Wl_ref[...],
                            preferred_element_type=jnp.float32) + bl_ref[...])


def kernel(x, edge_index, batch, W1a, b1a, g1, be1, W1b, b1b,
           W2a, b2a, g2, be2, W2b, b2b, Wl, bl):
    src = edge_index[0]
    dst = edge_index[1]

    agg1 = _sc_agg(x, src, dst)
    h1 = pl.pallas_call(
        _mlp_kernel,
        out_shape=jax.ShapeDtypeStruct((_N, _D), jnp.float32),
    )(x, agg1, W1a, b1a.reshape(1, -1), g1.reshape(1, -1),
      be1.reshape(1, -1), W1b, b1b.reshape(1, -1))

    agg2 = _sc_agg(h1, src, dst)
    out = pl.pallas_call(
        _mlp_pool_kernel,
        out_shape=jax.ShapeDtypeStruct((_G, 1), jnp.float32),
    )(h1, agg2, batch.reshape(1, -1), W2a, b2a.reshape(1, -1),
      g2.reshape(1, -1), be2.reshape(1, -1), W2b, b2b.reshape(1, -1),
      Wl, bl.reshape(1, -1))
    return out
